# Initial kernel scaffold; baseline (speedup 1.0000x reference)
import jax
import jax.numpy as jnp
from jax.experimental import pallas as pl


def kernel(x, W1, ln_gamma, ln_beta, W2_mlp, W2_mha):
    raise NotImplementedError("write your pallas kernel here")


# speedup vs baseline: 1.1012x; 1.1012x over previous
import jax
import jax.numpy as jnp
from jax.experimental import pallas as pl
from jax.experimental.pallas import tpu as pltpu

_EMBED_DIM = 1024
_MLP_DIM = 1024
_MHA_DIM = 128
_NEURONS = 4096
_HEADS = 16


def _fused_router_kernel(x_ref, w1_ref, w2m_ref, w2h_ref, neurons_ref, heads_ref):
    x = x_ref[...]
    out = jax.lax.dot_general(
        x, w1_ref[...], (((1,), (1,)), ((), ())),
        preferred_element_type=jnp.float32)
    mlp = out[:, :_MLP_DIM]
    mha = out[:, _MLP_DIM:]
    neurons_ref[...] = jax.lax.dot_general(
        mlp, w2m_ref[...], (((1,), (1,)), ((), ())),
        preferred_element_type=jnp.float32)
    heads_ref[...] = jax.lax.dot_general(
        mha, w2h_ref[...], (((1,), (1,)), ((), ())),
        preferred_element_type=jnp.float32)


def kernel(x, W1, ln_gamma, ln_beta, W2_mlp, W2_mha):
    del ln_gamma, ln_beta
    n_tok = x.shape[0]
    bt = 512
    grid = (n_tok // bt,)
    neurons, heads = pl.pallas_call(
        _fused_router_kernel,
        grid=grid,
        in_specs=[
            pl.BlockSpec((bt, _EMBED_DIM), lambda i: (i, 0)),
            pl.BlockSpec((_MLP_DIM + _MHA_DIM, _EMBED_DIM), lambda i: (0, 0)),
            pl.BlockSpec((_NEURONS, _MLP_DIM), lambda i: (0, 0)),
            pl.BlockSpec((_HEADS, _MHA_DIM), lambda i: (0, 0)),
        ],
        out_specs=[
            pl.BlockSpec((bt, _NEURONS), lambda i: (i, 0)),
            pl.BlockSpec((bt, _HEADS), lambda i: (i, 0)),
        ],
        out_shape=[
            jax.ShapeDtypeStruct((n_tok, _NEURONS), jnp.float32),
            jax.ShapeDtypeStruct((n_tok, _HEADS), jnp.float32),
        ],
        compiler_params=pltpu.CompilerParams(
            dimension_semantics=("arbitrary",)),
    )(x, W1, W2_mlp, W2_mha)
    return (neurons, heads)
